# baseline (speedup 1.0000x reference)
import functools

import jax
import jax.numpy as jnp
from jax import lax
from jax.experimental import pallas as pl
from jax.experimental.pallas import tpu as pltpu
from jax.experimental.pallas import tpu_sc as plsc

K_TOP = 64


def _encode_kernel(x_ref, w_ref, benc_ref, bdec_ref, z_ref):
    xc = x_ref[...] - bdec_ref[...]
    acc = jnp.dot(xc, w_ref[...], preferred_element_type=jnp.float32)
    z_ref[...] = jnp.maximum(acc + benc_ref[...], 0.0)


def _scalarize(v):
    if getattr(v, "ndim", 0) == 0:
        return v
    return jnp.min(v)


def _make_sc_threshold(n_tok, d_sae, k):
    info = plsc.get_sparse_core_info()
    nw = info.num_cores * info.num_subcores
    rows = n_tok // nw
    nvec = d_sae // 16
    mesh = plsc.VectorSubcoreMesh(core_axis_name="c", subcore_axis_name="s")

    @functools.partial(
        pl.kernel,
        mesh=mesh,
        out_type=jax.ShapeDtypeStruct((n_tok,), jnp.int32),
        scratch_types=[
            pltpu.VMEM((d_sae,), jnp.float32),
            pltpu.VMEM((2048,), jnp.float32),
            pltpu.VMEM((rows,), jnp.int32),
        ],
        compiler_params=pltpu.CompilerParams(needs_layout_passes=False),
    )
    def sc_thr(z_hbm, thr_hbm, row_v, hist_v, thr_v):
        wid = lax.axis_index("s") * info.num_cores + lax.axis_index("c")
        base = wid * rows
        ones16 = jnp.ones((16,), jnp.float32)
        zeros16 = jnp.zeros((16,), jnp.float32)
        lane0 = jnp.arange(16, dtype=jnp.int32) == 0

        def pick(counts_vec, needed):
            r = lax.rev(counts_vec, (0,))
            c = plsc.cumsum(r)
            mask = c >= needed
            l0 = _scalarize(plsc.all_reduce_ffs(mask))
            above = jnp.max(jnp.where(mask, 0.0, c))
            return 15 - l0, needed - above

        def level(nitems, vec_of, needed):
            star = jnp.int32(0)
            before = jnp.float32(0)
            carry = jnp.float32(0)
            for g in range(nitems - 1, -1, -1):
                s = jnp.sum(vec_of(g))
                found = jnp.logical_and(carry < needed, carry + s >= needed)
                star = jnp.where(found, jnp.int32(g), star)
                before = jnp.where(found, carry, before)
                carry = carry + s
            return star, needed - before

        def scan3(nvecs, needed):
            nsup = nvecs // 16

            def sup_sum(g):
                acc = hist_v[pl.ds(g * 256, 16)]
                for kk in range(1, 16):
                    acc = acc + hist_v[pl.ds(g * 256 + kk * 16, 16)]
                return acc

            g_star, needed = level(nsup, sup_sum, needed)
            k_star, needed = level(
                16, lambda kk: hist_v[pl.ds(g_star * 256 + kk * 16, 16)],
                needed)
            vec_ix = g_star * 16 + k_star
            l_star, needed = pick(hist_v[pl.ds(vec_ix * 16, 16)], needed)
            return vec_ix * 16 + l_star, needed

        def zero_hist(nvecs):
            for i in range(nvecs):
                hist_v[pl.ds(i * 16, 16)] = zeros16

        def row_body(r, _):
            pltpu.sync_copy(z_hbm.at[base + r], row_v)

            zero_hist(64)

            def body_a(j, _c):
                bits = lax.bitcast_convert_type(row_v[pl.ds(j * 16, 16)], jnp.int32)
                plsc.addupdate_scatter(
                    hist_v, [lax.shift_right_logical(bits, 21)], ones16)
                return 0

            lax.fori_loop(0, nvec, body_a, 0, unroll=8)
            b1, m1 = scan3(64, jnp.float32(k))

            zero_hist(64)

            def body_b(j, _c):
                bits = lax.bitcast_convert_type(row_v[pl.ds(j * 16, 16)], jnp.int32)
                msk = lax.shift_right_logical(bits, 21) == b1
                b2v = lax.shift_right_logical(bits, 11) & 0x3FF
                plsc.addupdate_scatter(hist_v, [b2v], ones16, mask=msk)
                return 0

            lax.fori_loop(0, nvec, body_b, 0, unroll=8)
            b2, m2 = scan3(64, m1)

            zero_hist(128)
            top21 = (b1 << 10) | b2

            def body_c(j, _c):
                bits = lax.bitcast_convert_type(row_v[pl.ds(j * 16, 16)], jnp.int32)
                msk = lax.shift_right_logical(bits, 11) == top21
                b3v = bits & 0x7FF
                plsc.addupdate_scatter(hist_v, [b3v], ones16, mask=msk)
                return 0

            lax.fori_loop(0, nvec, body_c, 0, unroll=8)
            b3, _ = scan3(128, m2)

            t_bits = (b1 << 21) | (b2 << 11) | b3
            plsc.store_scatter(
                thr_v, [jnp.full((16,), r, jnp.int32)],
                jnp.full((16,), t_bits, jnp.int32), mask=lane0)
            return 0

        lax.fori_loop(0, rows, row_body, 0)
        pltpu.sync_copy(thr_v, thr_hbm.at[pl.ds(base, rows)])

    return sc_thr


def _decode_kernel(z_ref, w_ref, bdec_ref, thr_ref, xrec_ref, zsp_ref):
    zs = jnp.where(z_ref[...] >= thr_ref[:, :1], z_ref[...], 0.0)
    zsp_ref[...] = zs
    j = pl.program_id(1)

    @pl.when(j == 0)
    def _init():
        xrec_ref[...] = jnp.broadcast_to(bdec_ref[...], xrec_ref.shape)

    xrec_ref[...] += jnp.dot(zs, w_ref[...],
                             preferred_element_type=jnp.float32)


def kernel(x, W_enc, b_enc, W_dec, b_dec):
    n_tok, d_in = x.shape
    d_sae = W_enc.shape[1]
    f32 = jnp.float32

    b_enc2 = b_enc.reshape(1, d_sae)
    b_dec2 = b_dec.reshape(1, d_in)

    tb = min(1024, n_tok)
    sb = min(1024, d_sae)
    nt, ns = n_tok // tb, d_sae // sb
    z = pl.pallas_call(
        _encode_kernel,
        grid=(ns, nt),
        in_specs=[
            pl.BlockSpec((tb, d_in), lambda j, i: (i, 0)),
            pl.BlockSpec((d_in, sb), lambda j, i: (0, j)),
            pl.BlockSpec((1, sb), lambda j, i: (0, j)),
            pl.BlockSpec((1, d_in), lambda j, i: (0, 0)),
        ],
        out_specs=pl.BlockSpec((tb, sb), lambda j, i: (i, j)),
        out_shape=jax.ShapeDtypeStruct((n_tok, d_sae), f32),
        compiler_params=pltpu.CompilerParams(
            dimension_semantics=("arbitrary", "arbitrary"),
        ),
    )(x, W_enc, b_enc2, b_dec2)

    thr_bits = _make_sc_threshold(n_tok, d_sae, K_TOP)(z)
    thr_f = jax.lax.bitcast_convert_type(thr_bits, f32)
    thr = jnp.broadcast_to(thr_f[:, None], (n_tok, 128))

    tb2 = min(1024, n_tok)
    kb2 = min(1024, d_sae)
    x_rec, z_sparse = pl.pallas_call(
        _decode_kernel,
        grid=(n_tok // tb2, d_sae // kb2),
        in_specs=[
            pl.BlockSpec((tb2, kb2), lambda i, j: (i, j)),
            pl.BlockSpec((kb2, d_in), lambda i, j: (j, 0)),
            pl.BlockSpec((1, d_in), lambda i, j: (0, 0)),
            pl.BlockSpec((tb2, 128), lambda i, j: (i, 0)),
        ],
        out_specs=[
            pl.BlockSpec((tb2, d_in), lambda i, j: (i, 0)),
            pl.BlockSpec((tb2, kb2), lambda i, j: (i, j)),
        ],
        out_shape=[
            jax.ShapeDtypeStruct((n_tok, d_in), f32),
            jax.ShapeDtypeStruct((n_tok, d_sae), f32),
        ],
        compiler_params=pltpu.CompilerParams(
            dimension_semantics=("parallel", "arbitrary"),
        ),
    )(z, W_dec, b_dec2, thr)

    return (x_rec, z_sparse)


# speedup vs baseline: 1.9380x; 1.9380x over previous
import functools

import jax
import jax.numpy as jnp
from jax import lax
from jax.experimental import pallas as pl
from jax.experimental.pallas import tpu as pltpu
from jax.experimental.pallas import tpu_sc as plsc

K_TOP = 64


def _encode_kernel(x_ref, w_ref, benc_ref, bdec_ref, z_ref):
    xc = x_ref[...] - bdec_ref[...]
    acc = jnp.dot(xc, w_ref[...], preferred_element_type=jnp.float32)
    z_ref[...] = jnp.maximum(acc + benc_ref[...], 0.0)


def _scalarize(v):
    if getattr(v, "ndim", 0) == 0:
        return v
    return jnp.min(v)


def _make_sc_threshold(n_tok, d_sae, k):
    info = plsc.get_sparse_core_info()
    nw = info.num_cores * info.num_subcores
    rows = n_tok // nw
    nvec = d_sae // 16
    mesh = plsc.VectorSubcoreMesh(core_axis_name="c", subcore_axis_name="s")

    @functools.partial(
        pl.kernel,
        mesh=mesh,
        out_type=jax.ShapeDtypeStruct((n_tok,), jnp.int32),
        scratch_types=[
            pltpu.VMEM((d_sae,), jnp.float32),
            pltpu.VMEM((d_sae,), jnp.float32),
            pltpu.VMEM((2048,), jnp.float32),
            pltpu.VMEM((rows,), jnp.int32),
            pltpu.SemaphoreType.DMA,
            pltpu.SemaphoreType.DMA,
        ],
        compiler_params=pltpu.CompilerParams(needs_layout_passes=False),
    )
    def sc_thr(z_hbm, thr_hbm, row0_v, row1_v, hist_v, thr_v, sem0, sem1):
        wid = lax.axis_index("s") * info.num_cores + lax.axis_index("c")
        base = wid * rows
        ones16 = jnp.ones((16,), jnp.float32)
        zeros16 = jnp.zeros((16,), jnp.float32)
        lane0 = jnp.arange(16, dtype=jnp.int32) == 0

        def pick(counts_vec, needed):
            r = lax.rev(counts_vec, (0,))
            c = plsc.cumsum(r)
            mask = c >= needed
            l0 = _scalarize(plsc.all_reduce_ffs(mask))
            above = jnp.max(jnp.where(mask, 0.0, c))
            return 15 - l0, needed - above

        def level(nitems, vec_of, needed):
            star = jnp.int32(0)
            before = jnp.float32(0)
            carry = jnp.float32(0)
            for g in range(nitems - 1, -1, -1):
                s = jnp.sum(vec_of(g))
                found = jnp.logical_and(carry < needed, carry + s >= needed)
                star = jnp.where(found, jnp.int32(g), star)
                before = jnp.where(found, carry, before)
                carry = carry + s
            return star, needed - before

        def scan3(nvecs, needed):
            nsup = nvecs // 16

            def sup_sum(g):
                acc = hist_v[pl.ds(g * 256, 16)]
                for kk in range(1, 16):
                    acc = acc + hist_v[pl.ds(g * 256 + kk * 16, 16)]
                return acc

            g_star, needed = level(nsup, sup_sum, needed)
            k_star, needed = level(
                16, lambda kk: hist_v[pl.ds(g_star * 256 + kk * 16, 16)],
                needed)
            vec_ix = g_star * 16 + k_star
            l_star, needed = pick(hist_v[pl.ds(vec_ix * 16, 16)], needed)
            return vec_ix * 16 + l_star, needed

        def zero_hist(nvecs):
            for i in range(nvecs):
                hist_v[pl.ds(i * 16, 16)] = zeros16

        def radix_row(row_ref):
            zero_hist(64)

            @plsc.parallel_loop(0, nvec, unroll=8)
            def _pa(j):
                bits = lax.bitcast_convert_type(
                    row_ref[pl.ds(j * 16, 16)], jnp.int32)
                plsc.addupdate_scatter(
                    hist_v, [lax.shift_right_logical(bits, 21)], ones16)

            b1, m1 = scan3(64, jnp.float32(k))

            zero_hist(64)

            @plsc.parallel_loop(0, nvec, unroll=8)
            def _pb(j):
                bits = lax.bitcast_convert_type(
                    row_ref[pl.ds(j * 16, 16)], jnp.int32)
                msk = lax.shift_right_logical(bits, 21) == b1
                b2v = lax.shift_right_logical(bits, 11) & 0x3FF
                plsc.addupdate_scatter(hist_v, [b2v], ones16, mask=msk)

            b2, m2 = scan3(64, m1)

            zero_hist(128)
            top21 = (b1 << 10) | b2

            @plsc.parallel_loop(0, nvec, unroll=8)
            def _pc(j):
                bits = lax.bitcast_convert_type(
                    row_ref[pl.ds(j * 16, 16)], jnp.int32)
                msk = lax.shift_right_logical(bits, 11) == top21
                b3v = bits & 0x7FF
                plsc.addupdate_scatter(hist_v, [b3v], ones16, mask=msk)

            b3, _ = scan3(128, m2)
            return (b1 << 21) | (b2 << 11) | b3

        def put_thr(r, t_bits):
            plsc.store_scatter(
                thr_v, [jnp.full((16,), r, jnp.int32)],
                jnp.full((16,), t_bits, jnp.int32), mask=lane0)

        pltpu.async_copy(z_hbm.at[base], row0_v, sem0)

        def pair_body(p, _):
            r0 = base + 2 * p
            pltpu.async_copy(z_hbm.at[r0 + 1], row1_v, sem1)
            pltpu.make_async_copy(z_hbm.at[r0], row0_v, sem0).wait()
            put_thr(2 * p, radix_row(row0_v))

            @pl.when(2 * p + 2 < rows)
            def _prefetch():
                pltpu.async_copy(z_hbm.at[r0 + 2], row0_v, sem0)

            pltpu.make_async_copy(z_hbm.at[r0 + 1], row1_v, sem1).wait()
            put_thr(2 * p + 1, radix_row(row1_v))
            return 0

        lax.fori_loop(0, rows // 2, pair_body, 0)
        pltpu.sync_copy(thr_v, thr_hbm.at[pl.ds(base, rows)])

    return sc_thr


def _decode_kernel(z_ref, w_ref, bdec_ref, thr_ref, xrec_ref, zsp_ref):
    zs = jnp.where(z_ref[...] >= thr_ref[:, :1], z_ref[...], 0.0)
    zsp_ref[...] = zs
    j = pl.program_id(1)

    @pl.when(j == 0)
    def _init():
        xrec_ref[...] = jnp.broadcast_to(bdec_ref[...], xrec_ref.shape)

    xrec_ref[...] += jnp.dot(zs, w_ref[...],
                             preferred_element_type=jnp.float32)


def kernel(x, W_enc, b_enc, W_dec, b_dec):
    n_tok, d_in = x.shape
    d_sae = W_enc.shape[1]
    f32 = jnp.float32

    b_enc2 = b_enc.reshape(1, d_sae)
    b_dec2 = b_dec.reshape(1, d_in)

    tb = min(1024, n_tok)
    sb = min(1024, d_sae)
    nt, ns = n_tok // tb, d_sae // sb
    z = pl.pallas_call(
        _encode_kernel,
        grid=(ns, nt),
        in_specs=[
            pl.BlockSpec((tb, d_in), lambda j, i: (i, 0)),
            pl.BlockSpec((d_in, sb), lambda j, i: (0, j)),
            pl.BlockSpec((1, sb), lambda j, i: (0, j)),
            pl.BlockSpec((1, d_in), lambda j, i: (0, 0)),
        ],
        out_specs=pl.BlockSpec((tb, sb), lambda j, i: (i, j)),
        out_shape=jax.ShapeDtypeStruct((n_tok, d_sae), f32),
        compiler_params=pltpu.CompilerParams(
            dimension_semantics=("arbitrary", "arbitrary"),
        ),
    )(x, W_enc, b_enc2, b_dec2)

    thr_bits = _make_sc_threshold(n_tok, d_sae, K_TOP)(z)
    thr_f = jax.lax.bitcast_convert_type(thr_bits, f32)
    thr = jnp.broadcast_to(thr_f[:, None], (n_tok, 128))

    tb2 = min(1024, n_tok)
    kb2 = min(1024, d_sae)
    x_rec, z_sparse = pl.pallas_call(
        _decode_kernel,
        grid=(n_tok // tb2, d_sae // kb2),
        in_specs=[
            pl.BlockSpec((tb2, kb2), lambda i, j: (i, j)),
            pl.BlockSpec((kb2, d_in), lambda i, j: (j, 0)),
            pl.BlockSpec((1, d_in), lambda i, j: (0, 0)),
            pl.BlockSpec((tb2, 128), lambda i, j: (i, 0)),
        ],
        out_specs=[
            pl.BlockSpec((tb2, d_in), lambda i, j: (i, 0)),
            pl.BlockSpec((tb2, kb2), lambda i, j: (i, j)),
        ],
        out_shape=[
            jax.ShapeDtypeStruct((n_tok, d_in), f32),
            jax.ShapeDtypeStruct((n_tok, d_sae), f32),
        ],
        compiler_params=pltpu.CompilerParams(
            dimension_semantics=("parallel", "arbitrary"),
        ),
    )(z, W_dec, b_dec2, thr)

    return (x_rec, z_sparse)
